# baseline (speedup 1.0000x reference)
import functools

import jax
import jax.numpy as jnp
from jax import lax
from jax.experimental import pallas as pl
from jax.experimental.pallas import tpu as pltpu
from jax.experimental.pallas import tpu_sc as plsc

_VOCAB = 100000
_D = 128
_HID = 1024
_NCLS = 1000
_B = 16384
_HIST = 20

_NSLICES = 4
_BS = _B // _NSLICES

_NC = 2
_NS = 16
_NW = _NC * _NS
_SB = 32
_RB = _SB * _HIST
_NG = _RB // 128

_SMAX = [(128 * (g + 1)) // _HIST for g in range(_NG)]

@functools.cache
def _make_sc_pool(nsamp, slice_off):
    bpw = nsamp // _NW
    ipw = bpw * _HIST
    nstep = bpw // _SB
    mesh = plsc.VectorSubcoreMesh(core_axis_name="c", subcore_axis_name="s")

    @functools.partial(
        pl.kernel,
        mesh=mesh,
        out_type=jax.ShapeDtypeStruct((nsamp, _D), jnp.float32),
        scratch_types=[
            pltpu.VMEM((ipw,), jnp.int32),
            pltpu.VMEM((_RB, _D), jnp.float32),
            pltpu.VMEM((_SB, _D), jnp.float32),
        ] + [pltpu.SemaphoreType.DMA] * _NG,
    )
    def sc_pool(idx_hbm, table_hbm, out_hbm, idx_v, rows_v, out_v, *sems):
        wid = lax.axis_index("s") * _NC + lax.axis_index("c")
        pltpu.sync_copy(idx_hbm.at[pl.ds(slice_off + wid * ipw, ipw)], idx_v)

        def gcp(t, g):
            return pltpu.make_async_copy(
                table_hbm.at[idx_v.at[pl.ds((t * _NG + g) * 128, 128)]],
                rows_v.at[pl.ds(g * 128, 128)],
                sems[g],
            )

        for g in range(_NG):
            gcp(0, g).start()

        def sample(i, c2):
            r0 = i * _HIST
            for gg in range(_D // 16):
                sl = pl.ds(gg * 16, 16)
                acc = rows_v[r0, sl]
                for j in range(1, _HIST):
                    acc = acc + rows_v[r0 + j, sl]
                out_v[i, sl] = acc
            return c2

        def step(t, carry):
            for g in range(_NG):
                gcp(t, g).wait()
                lo = 0 if g == 0 else _SMAX[g - 1]
                lax.fori_loop(lo, _SMAX[g], sample, 0)
                if g >= 1:
                    @pl.when(t < nstep - 1)
                    def _():
                        gcp(t + 1, g - 1).start()

            @pl.when(t < nstep - 1)
            def _():
                gcp(t + 1, _NG - 1).start()

            pltpu.sync_copy(out_v, out_hbm.at[pl.ds(wid * bpw + t * _SB, _SB)])
            return carry

        lax.fori_loop(0, nstep, step, 0)

    return sc_pool


def _mlp_body(x_ref, bias_ref, w1_ref, b1_ref, w2_ref, b2_ref, w3_ref,
              b3_ref, *rest):
    out_ref = rest[-1]
    x = (x_ref[...] + bias_ref[...]).astype(jnp.bfloat16)
    h = jnp.tanh(
        jnp.dot(x, w1_ref[...], preferred_element_type=jnp.float32)
        + b1_ref[...])
    h = jnp.tanh(
        jnp.dot(h.astype(jnp.bfloat16), w2_ref[...],
                preferred_element_type=jnp.float32)
        + b2_ref[...])
    out_ref[...] = (
        lax.dot_general(w3_ref[...], h.astype(jnp.bfloat16),
                        (((0,), (1,)), ((), ())),
                        preferred_element_type=jnp.float32)
        + b3_ref[...])


_TB = 512


def _mlp_slice(k, pooled, bias, W1, b1, W2, b2, W3, b3, buf):
    ntile = _BS // _TB
    full = lambda shape: pl.BlockSpec(shape, lambda i: (0, 0))
    in_specs = [
        pl.BlockSpec((_TB, _D), lambda i: (i, 0)),
        full((1, _D)),
        full((_D, _HID)),
        full((1, _HID)),
        full((_HID, _HID)),
        full((1, _HID)),
        full((_HID, _NCLS)),
        full((_NCLS, 1)),
    ]
    args = [pooled, bias, W1, b1, W2, b2, W3, b3]
    aliases = {}
    if buf is not None:
        in_specs.append(pl.BlockSpec(memory_space=pl.ANY))
        args.append(buf)
        aliases = {8: 0}
    return pl.pallas_call(
        _mlp_body,
        grid=(ntile,),
        in_specs=in_specs,
        out_specs=pl.BlockSpec((_NCLS, _TB), lambda i, _k=k: (0, i + _k * ntile)),
        out_shape=jax.ShapeDtypeStruct((_NCLS, _B), jnp.float32),
        input_output_aliases=aliases,
    )(*args)


def kernel(inputs, table, bias, W1, b1, W2, b2, W3, b3):
    idx = inputs.reshape(_B * _HIST).astype(jnp.int32)
    bias2 = bias.reshape(1, _D)
    w1b = W1.astype(jnp.bfloat16)
    w2b = W2.astype(jnp.bfloat16)
    w3b = W3.astype(jnp.bfloat16)
    b1r = b1.reshape(1, _HID)
    b2r = b2.reshape(1, _HID)
    b3r = b3.reshape(_NCLS, 1)
    buf = None
    for k in range(_NSLICES):
        sc_pool = _make_sc_pool(_BS, k * _BS * _HIST)
        pooled_k = sc_pool(idx, table)
        buf = _mlp_slice(k, pooled_k, bias2, w1b, b1r, w2b, b2r, w3b, b3r,
                         buf)
    return buf.T


# speedup vs baseline: 1.0632x; 1.0632x over previous
import functools

import jax
import jax.numpy as jnp
from jax import lax
from jax.experimental import pallas as pl
from jax.experimental.pallas import tpu as pltpu
from jax.experimental.pallas import tpu_sc as plsc

_VOCAB = 100000
_D = 128
_HID = 1024
_NCLS = 1000
_B = 16384
_HIST = 20

_NSLICES = 4
_BS = _B // _NSLICES

_NC = 2
_NS = 16
_NW = _NC * _NS
_SB = 32
_RB = _SB * _HIST
_NG = _RB // 128

_SMAX = [(128 * (g + 1)) // _HIST for g in range(_NG)]

@functools.cache
def _make_sc_pool(nsamp, slice_off):
    bpw = nsamp // _NW
    ipw = bpw * _HIST
    nstep = bpw // _SB
    mesh = plsc.VectorSubcoreMesh(core_axis_name="c", subcore_axis_name="s")

    @functools.partial(
        pl.kernel,
        mesh=mesh,
        out_type=jax.ShapeDtypeStruct((nsamp, _D), jnp.float32),
        scratch_types=[
            pltpu.VMEM((ipw,), jnp.int32),
            pltpu.VMEM((_RB, _D), jnp.float32),
            pltpu.VMEM((_SB, _D), jnp.float32),
        ] + [pltpu.SemaphoreType.DMA] * _NG,
    )
    def sc_pool(idx_hbm, table_hbm, out_hbm, idx_v, rows_v, out_v, *sems):
        wid = lax.axis_index("s") * _NC + lax.axis_index("c")
        pltpu.sync_copy(idx_hbm.at[pl.ds(slice_off + wid * ipw, ipw)], idx_v)

        def gcp(t, g):
            return pltpu.make_async_copy(
                table_hbm.at[idx_v.at[pl.ds((t * _NG + g) * 128, 128)]],
                rows_v.at[pl.ds(g * 128, 128)],
                sems[g],
            )

        for g in range(_NG):
            gcp(0, g).start()

        def sample(i, c2):
            r0 = i * _HIST
            for gg in range(_D // 16):
                sl = pl.ds(gg * 16, 16)
                acc = rows_v[r0, sl]
                for j in range(1, _HIST):
                    acc = acc + rows_v[r0 + j, sl]
                out_v[i, sl] = acc
            return c2

        def step(t, carry):
            for g in range(_NG):
                gcp(t, g).wait()
                lo = 0 if g == 0 else _SMAX[g - 1]
                lax.fori_loop(lo, _SMAX[g], sample, 0)
                if g >= 1:
                    @pl.when(t < nstep - 1)
                    def _():
                        gcp(t + 1, g - 1).start()

            @pl.when(t < nstep - 1)
            def _():
                gcp(t + 1, _NG - 1).start()

            pltpu.sync_copy(out_v, out_hbm.at[pl.ds(wid * bpw + t * _SB, _SB)])
            return carry

        lax.fori_loop(0, nstep, step, 0)

    return sc_pool


def _mlp_body(x_ref, bias_ref, w1_ref, b1_ref, w2_ref, b2_ref, w3_ref,
              b3_ref, *rest):
    out_ref = rest[-1]
    x = (x_ref[...] + bias_ref[...]).astype(jnp.bfloat16)
    h = jnp.tanh(
        jnp.dot(x, w1_ref[...], preferred_element_type=jnp.float32)
        + b1_ref[...])
    h = jnp.tanh(
        jnp.dot(h.astype(jnp.bfloat16), w2_ref[...],
                preferred_element_type=jnp.float32)
        + b2_ref[...])
    out_ref[...] = (
        lax.dot_general(w3_ref[...], h.astype(jnp.bfloat16),
                        (((0,), (1,)), ((), ())),
                        preferred_element_type=jnp.float32)
        + b3_ref[...])


_TB = 512


def _mlp_slice(k, pooled, bias, W1, b1, W2, b2, W3, b3, buf):
    ntile = _BS // _TB
    full = lambda shape: pl.BlockSpec(shape, lambda i: (0, 0))
    in_specs = [
        pl.BlockSpec((_TB, _D), lambda i: (i, 0)),
        full((1, _D)),
        full((_D, _HID)),
        full((1, _HID)),
        full((_HID, _HID)),
        full((1, _HID)),
        full((_HID, _NCLS)),
        full((_NCLS, 1)),
    ]
    args = [pooled, bias, W1, b1, W2, b2, W3, b3]
    aliases = {}
    if buf is not None:
        in_specs.append(pl.BlockSpec(memory_space=pl.ANY))
        args.append(buf)
        aliases = {8: 0}
    return pl.pallas_call(
        _mlp_body,
        grid=(ntile,),
        in_specs=in_specs,
        out_specs=pl.BlockSpec((_NCLS, _TB), lambda i, _k=k: (0, i + _k * ntile)),
        out_shape=jax.ShapeDtypeStruct((_NCLS, _B), jnp.float32),
        input_output_aliases=aliases,
    )(*args)


def kernel(inputs, table, bias, W1, b1, W2, b2, W3, b3):
    bias2 = bias.reshape(1, _D)
    w1b = W1.astype(jnp.bfloat16)
    w2b = W2.astype(jnp.bfloat16)
    w3b = W3.astype(jnp.bfloat16)
    b1r = b1.reshape(1, _HID)
    b2r = b2.reshape(1, _HID)
    b3r = b3.reshape(_NCLS, 1)
    buf = None
    for k in range(_NSLICES):
        sc_pool = _make_sc_pool(_BS, 0)
        idx_k = inputs[k * _BS:(k + 1) * _BS].reshape(_BS * _HIST)
        pooled_k = sc_pool(idx_k.astype(jnp.int32), table)
        buf = _mlp_slice(k, pooled_k, bias2, w1b, b1r, w2b, b2r, w3b, b3r,
                         buf)
    return buf.T


# speedup vs baseline: 1.0982x; 1.0329x over previous
import functools

import jax
import jax.numpy as jnp
from jax import lax
from jax.experimental import pallas as pl
from jax.experimental.pallas import tpu as pltpu
from jax.experimental.pallas import tpu_sc as plsc

_VOCAB = 100000
_D = 128
_HID = 1024
_NCLS = 1000
_B = 16384
_HIST = 20

_SLICES = (2048, 6144, 6144, 2048)

_NC = 2
_NS = 16
_NW = _NC * _NS
_SB = 32
_RB = _SB * _HIST
_NG = _RB // 128

_SMAX = [(128 * (g + 1)) // _HIST for g in range(_NG)]

@functools.cache
def _make_sc_pool(nsamp, slice_off):
    bpw = nsamp // _NW
    ipw = bpw * _HIST
    nstep = bpw // _SB
    mesh = plsc.VectorSubcoreMesh(core_axis_name="c", subcore_axis_name="s")

    @functools.partial(
        pl.kernel,
        mesh=mesh,
        out_type=jax.ShapeDtypeStruct((nsamp, _D), jnp.float32),
        scratch_types=[
            pltpu.VMEM((ipw,), jnp.int32),
            pltpu.VMEM((_RB, _D), jnp.float32),
            pltpu.VMEM((_SB, _D), jnp.float32),
        ] + [pltpu.SemaphoreType.DMA] * _NG,
    )
    def sc_pool(idx_hbm, table_hbm, out_hbm, idx_v, rows_v, out_v, *sems):
        wid = lax.axis_index("s") * _NC + lax.axis_index("c")
        pltpu.sync_copy(idx_hbm.at[pl.ds(slice_off + wid * ipw, ipw)], idx_v)

        def gcp(t, g):
            return pltpu.make_async_copy(
                table_hbm.at[idx_v.at[pl.ds((t * _NG + g) * 128, 128)]],
                rows_v.at[pl.ds(g * 128, 128)],
                sems[g],
            )

        for g in range(_NG):
            gcp(0, g).start()

        def sample(i, c2):
            r0 = i * _HIST
            for gg in range(_D // 16):
                sl = pl.ds(gg * 16, 16)
                acc = rows_v[r0, sl]
                for j in range(1, _HIST):
                    acc = acc + rows_v[r0 + j, sl]
                out_v[i, sl] = acc
            return c2

        def step(t, carry):
            for g in range(_NG):
                gcp(t, g).wait()
                lo = 0 if g == 0 else _SMAX[g - 1]
                lax.fori_loop(lo, _SMAX[g], sample, 0)
                if g >= 1:
                    @pl.when(t < nstep - 1)
                    def _():
                        gcp(t + 1, g - 1).start()

            @pl.when(t < nstep - 1)
            def _():
                gcp(t + 1, _NG - 1).start()

            pltpu.sync_copy(out_v, out_hbm.at[pl.ds(wid * bpw + t * _SB, _SB)])
            return carry

        lax.fori_loop(0, nstep, step, 0)

    return sc_pool


def _mlp_body(x_ref, bias_ref, w1_ref, b1_ref, w2_ref, b2_ref, w3_ref,
              b3_ref, *rest):
    out_ref = rest[-1]
    x = (x_ref[...] + bias_ref[...]).astype(jnp.bfloat16)
    h = jnp.tanh(
        jnp.dot(x, w1_ref[...], preferred_element_type=jnp.float32)
        + b1_ref[...])
    h = jnp.tanh(
        jnp.dot(h.astype(jnp.bfloat16), w2_ref[...],
                preferred_element_type=jnp.float32)
        + b2_ref[...])
    out_ref[...] = (
        lax.dot_general(w3_ref[...], h.astype(jnp.bfloat16),
                        (((0,), (1,)), ((), ())),
                        preferred_element_type=jnp.float32)
        + b3_ref[...])


_TB = 512


def _mlp_slice(tile_off, pooled, bias, W1, b1, W2, b2, W3, b3, buf):
    ntile = pooled.shape[0] // _TB
    full = lambda shape: pl.BlockSpec(shape, lambda i: (0, 0))
    in_specs = [
        pl.BlockSpec((_TB, _D), lambda i: (i, 0)),
        full((1, _D)),
        full((_D, _HID)),
        full((1, _HID)),
        full((_HID, _HID)),
        full((1, _HID)),
        full((_HID, _NCLS)),
        full((_NCLS, 1)),
    ]
    args = [pooled, bias, W1, b1, W2, b2, W3, b3]
    aliases = {}
    if buf is not None:
        in_specs.append(pl.BlockSpec(memory_space=pl.ANY))
        args.append(buf)
        aliases = {8: 0}
    return pl.pallas_call(
        _mlp_body,
        grid=(ntile,),
        in_specs=in_specs,
        out_specs=pl.BlockSpec((_NCLS, _TB),
                               lambda i, _o=tile_off: (0, i + _o)),
        out_shape=jax.ShapeDtypeStruct((_NCLS, _B), jnp.float32),
        input_output_aliases=aliases,
    )(*args)


def kernel(inputs, table, bias, W1, b1, W2, b2, W3, b3):
    bias2 = bias.reshape(1, _D)
    w1b = W1.astype(jnp.bfloat16)
    w2b = W2.astype(jnp.bfloat16)
    w3b = W3.astype(jnp.bfloat16)
    b1r = b1.reshape(1, _HID)
    b2r = b2.reshape(1, _HID)
    b3r = b3.reshape(_NCLS, 1)
    buf = None
    base = 0
    for bs in _SLICES:
        sc_pool = _make_sc_pool(bs, 0)
        idx_k = inputs[base:base + bs].reshape(bs * _HIST)
        pooled_k = sc_pool(idx_k.astype(jnp.int32), table)
        buf = _mlp_slice(base // _TB, pooled_k, bias2, w1b, b1r, w2b, b2r,
                         w3b, b3r, buf)
        base += bs
    return buf.T
